# baseline (speedup 1.0000x reference)
import dataclasses

import jax
import jax.numpy as jnp
from jax import lax
from jax.experimental import pallas as pl
from jax.experimental.pallas import tpu as pltpu
from jax.experimental.pallas import tpu_sc as plsc

_N = 10000
_E = 320000
_D = 128
_NC = 2
_NS = 16
_NW = _NC * _NS
_CH = 160
_NBW = 64
_EPW = _NBW * _CH
_NPAD = 10240
_ZR = _NPAD // _NS
_TRASH = _NPAD - 2


def _sc_compiler_params():
    cp = pltpu.CompilerParams()
    if "needs_layout_passes" in pltpu.CompilerParams.__dataclass_fields__:
        cp = dataclasses.replace(cp, needs_layout_passes=False)
    return cp


def _sc_edge_pass(srcp, dstp, asrc, adst, mvec, h, zrows, zden):
    mesh = plsc.VectorSubcoreMesh(core_axis_name="c", subcore_axis_name="s")

    @pl.kernel(
        out_type=[
            jax.ShapeDtypeStruct((_NC, _N, _D), jnp.float32),
            jax.ShapeDtypeStruct((_NC, _NPAD), jnp.float32),
        ],
        mesh=mesh,
        scratch_types=[
            pltpu.VMEM((16,), jnp.float32),
            pltpu.VMEM((_CH,), jnp.int32),
            pltpu.VMEM((_CH,), jnp.int32),
            pltpu.VMEM((_CH,), jnp.int32),
            pltpu.VMEM((_CH,), jnp.int32),
            pltpu.VMEM((_CH,), jnp.float32),
            pltpu.VMEM((_CH,), jnp.float32),
            pltpu.VMEM((_CH,), jnp.float32),
            pltpu.VMEM((_CH,), jnp.float32),
            pltpu.VMEM((_CH, _D), jnp.float32),
            pltpu.VMEM((_CH, _D), jnp.float32),
            pltpu.VMEM_SHARED((_NPAD, _D), jnp.float32),
            pltpu.VMEM_SHARED((_NPAD,), jnp.float32),
            pltpu.VMEM_SHARED((_NPAD,), jnp.float32),
            pltpu.VMEM_SHARED((_NPAD,), jnp.float32),
            pltpu.SemaphoreType.DMA,
            pltpu.SemaphoreType.DMA,
            pltpu.SemaphoreType.DMA,
            pltpu.SemaphoreType.DMA,
            pltpu.SemaphoreType.DMA,
        ],
        compiler_params=_sc_compiler_params(),
    )
    def edge_kernel(src_r, dst_r, asc_r, adc_r, m_r, h_r, zr_r, zd_r,
                    accp_r, denp_r,
                    m_t, srcb0, srcb1, dstb0, dstb1, pb0, pb1, av, bv,
                    rows0, rows1, acc_sp, den_sp, asc_sp, adc_sp,
                    gsem0, gsem1, ssem0, ssem1, psem):
        c = lax.axis_index("c")
        s = lax.axis_index("s")
        w = c * _NS + s
        base_w = w * _EPW

        pltpu.sync_copy(m_r, m_t)

        pltpu.sync_copy(zr_r, acc_sp.at[pl.ds(s * _ZR, _ZR)])
        pltpu.sync_copy(zd_r, den_sp.at[pl.ds(s * _ZR, _ZR)])

        @pl.when(s == 0)
        def _stage_scores():
            pltpu.sync_copy(asc_r, asc_sp)
            pltpu.sync_copy(adc_r, adc_sp)

        plsc.subcore_barrier()

        mv = m_t[...]
        slots = ((srcb0, dstb0, pb0, rows0, gsem0, ssem0),
                 (srcb1, dstb1, pb1, rows1, gsem1, ssem1))

        def idx_load(i, sl):
            sT, dT = slots[sl][0], slots[sl][1]
            pltpu.sync_copy(src_r.at[pl.ds(base_w + i * _CH, _CH)], sT)
            pltpu.sync_copy(dst_r.at[pl.ds(base_w + i * _CH, _CH)], dT)

        def scores_start(sl):
            sT, dT = slots[sl][0], slots[sl][1]
            pltpu.async_copy(asc_sp.at[sT], av, psem)
            pltpu.async_copy(adc_sp.at[dT], bv, psem)

        def pb_compute(sl):
            pltpu.make_async_copy(asc_r.at[pl.ds(0, _CH)], av, psem).wait()
            pltpu.make_async_copy(asc_r.at[pl.ds(0, _CH)], bv, psem).wait()
            pT = slots[sl][2]

            @pl.loop(0, _CH, step=16)
            def _group(j):
                e = av[pl.ds(j, 16)] + bv[pl.ds(j, 16)]
                e = jnp.where(e >= 0.0, e, e * 0.2)
                pT[pl.ds(j, 16)] = jnp.exp(e - mv)

        def gather_start(sl):
            sT, rT, gT = slots[sl][0], slots[sl][3], slots[sl][4]
            pltpu.async_copy(h_r.at[sT], rT, gT)

        def body(i, sl, first=False, last=False):
            sT, dT, pT, rT, gT, ssT = slots[sl]
            pltpu.make_async_copy(h_r.at[pl.ds(0, _CH)], rT, gT).wait()

            @pl.loop(0, _CH, step=4)
            def _scale(rr):
                for u in range(4):
                    pv = plsc.load_gather(
                        pT, [jnp.broadcast_to(rr + u, (16,))])
                    for k in range(0, _D, 16):
                        rT[rr + u, pl.ds(k, 16)] = \
                            rT[rr + u, pl.ds(k, 16)] * pv

            pltpu.async_copy(rT, acc_sp.at[dT], ssT, add=True)
            pltpu.sync_copy(pT, den_sp.at[dT], add=True)

            if not first:
                oT = slots[1 - sl]
                pltpu.make_async_copy(h_r.at[pl.ds(0, _CH)], oT[3],
                                      oT[5]).wait()
            if not last:
                idx_load(i + 1, 1 - sl)
                scores_start(1 - sl)
                gather_start(1 - sl)
                pb_compute(1 - sl)

        idx_load(0, 0)
        scores_start(0)
        gather_start(0)
        pb_compute(0)

        body(0, 0, first=True)

        @pl.loop(0, (_NBW - 2) // 2, step=1)
        def _steady(k):
            body(2 * k + 1, 1)
            body(2 * k + 2, 0)

        body(_NBW - 1, 1, last=True)
        pltpu.make_async_copy(h_r.at[pl.ds(0, _CH)], rows1, ssem1).wait()

        plsc.subcore_barrier()

        @pl.when(s == 0)
        def _writeback():
            pltpu.sync_copy(acc_sp.at[pl.ds(0, _N)], accp_r.at[c])
            pltpu.sync_copy(den_sp, denp_r.at[c])

    return edge_kernel(srcp, dstp, asrc, adst, mvec, h, zrows, zden)


def _attn_tail(h, a_s, a_d, as_o, ad_o, m_o):
    asr = jnp.sum(h * a_s[None, :], axis=1)
    adr = jnp.sum(h * a_d[None, :], axis=1)
    zpad = jnp.zeros((_NPAD - _N,), jnp.float32)
    as_o[...] = jnp.concatenate([asr, zpad])
    ad_o[...] = jnp.concatenate([adr, zpad])
    m = jnp.max(asr) + jnp.max(adr)
    m = jnp.where(m >= 0.0, m, m * 0.2)
    m_o[...] = jnp.broadcast_to(m, (16,))


_TC_OUT_TYPES = [
    jax.ShapeDtypeStruct((_N, _D), jnp.float32),
    jax.ShapeDtypeStruct((_NPAD,), jnp.float32),
    jax.ShapeDtypeStruct((_NPAD,), jnp.float32),
    jax.ShapeDtypeStruct((16,), jnp.float32),
]


def _tc_in(x, W1, a_s, a_d):
    def body(x_r, w_r, as_r, ad_r, h_o, as_o, ad_o, m_o):
        h = lax.dot_general(x_r[...], w_r[...], (((1,), (1,)), ((), ())),
                            preferred_element_type=jnp.float32)
        h_o[...] = h
        _attn_tail(h, as_r[...], ad_r[...], as_o, ad_o, m_o)

    return pl.pallas_call(body, out_shape=_TC_OUT_TYPES)(x, W1, a_s, a_d)


def _norm_relu(acc_r, den_r, b):
    den = den_r[0, : _N] + den_r[1, : _N] + 1e-16
    hs = acc_r[0] + acc_r[1]
    h = hs / den[:, None] + b[None, :]
    return jnp.maximum(h, 0.0)


def _tc_mid(accp, denp, b, W, a_s, a_d):
    def body(acc_r, den_r, b_r, w_r, as_r, ad_r, h_o, as_o, ad_o, m_o):
        h1 = _norm_relu(acc_r, den_r, b_r[...])
        h = lax.dot_general(h1, w_r[...], (((1,), (1,)), ((), ())),
                            preferred_element_type=jnp.float32)
        h_o[...] = h
        _attn_tail(h, as_r[...], ad_r[...], as_o, ad_o, m_o)

    return pl.pallas_call(body, out_shape=_TC_OUT_TYPES)(
        accp, denp, b, W, a_s, a_d)


def _tc_out(accp, denp, b, fsW, fsb):
    def body(acc_r, den_r, b_r, w_r, wb_r, o_r):
        h = _norm_relu(acc_r, den_r, b_r[...])
        o_r[...] = lax.dot_general(h, w_r[...], (((1,), (1,)), ((), ())),
                                   preferred_element_type=jnp.float32) \
            + wb_r[...][None, :]

    return pl.pallas_call(
        body,
        out_shape=[jax.ShapeDtypeStruct((_N, 16), jnp.float32)],
    )(accp, denp, b, fsW, fsb)[0]


def kernel(x, edge_index, W1, att_src1, att_dst1, b1,
           W2, att_src2, att_dst2, b2, fW, fb, sW, sb):
    nfake = _EPW - _E // _NW
    src2 = edge_index[0].astype(jnp.int32).reshape(_NW, _E // _NW)
    dst2 = edge_index[1].astype(jnp.int32).reshape(_NW, _E // _NW)
    srcp = jnp.concatenate(
        [src2, jnp.zeros((_NW, nfake), jnp.int32)], axis=1).reshape(-1)
    dstp = jnp.concatenate(
        [dst2, jnp.full((_NW, nfake), _TRASH, jnp.int32)], axis=1).reshape(-1)
    zrows = jnp.zeros((_ZR, _D), jnp.float32)
    zden = jnp.zeros((_ZR,), jnp.float32)

    h1, as1, ad1, m1 = _tc_in(x, W1, att_src1, att_dst1)
    acc1, den1 = _sc_edge_pass(srcp, dstp, as1, ad1, m1, h1, zrows, zden)
    h2, as2, ad2, m2 = _tc_mid(acc1, den1, b1, W2, att_src2, att_dst2)
    acc2, den2 = _sc_edge_pass(srcp, dstp, as2, ad2, m2, h2, zrows, zden)

    fsW = jnp.concatenate([fW, sW, jnp.zeros((6, _D), jnp.float32)], axis=0)
    fsb = jnp.concatenate([fb, sb, jnp.zeros((6,), jnp.float32)], axis=0)
    out = _tc_out(acc2, den2, b2, fsW, fsb)
    return out[:, :3], out[:, 3:10]


# speedup vs baseline: 1.7053x; 1.7053x over previous
import dataclasses

import jax
import jax.numpy as jnp
from jax import lax
from jax.experimental import pallas as pl
from jax.experimental.pallas import tpu as pltpu
from jax.experimental.pallas import tpu_sc as plsc

_N = 10000
_E = 320000
_D = 128
_NC = 2
_NS = 16
_NW = _NC * _NS
_CH = 208
_EP = 320320
_NB = _EP // _CH
_NPAD = 10240
_ZR = _NPAD // _NS
_TRASH = _NPAD - 2


def _sc_compiler_params():
    cp = pltpu.CompilerParams()
    if "needs_layout_passes" in pltpu.CompilerParams.__dataclass_fields__:
        cp = dataclasses.replace(cp, needs_layout_passes=False)
    return cp


def _sc_edge_pass(src, dst, asrc, adst, mvec, h, zrows, zden):
    mesh = plsc.VectorSubcoreMesh(core_axis_name="c", subcore_axis_name="s")

    @pl.kernel(
        out_type=[
            jax.ShapeDtypeStruct((_NC, _N, _D), jnp.float32),
            jax.ShapeDtypeStruct((_NC, _NPAD), jnp.float32),
        ],
        mesh=mesh,
        scratch_types=[
            pltpu.VMEM((_N,), jnp.float32),
            pltpu.VMEM((_N,), jnp.float32),
            pltpu.VMEM((16,), jnp.float32),
            pltpu.VMEM((_CH,), jnp.int32),
            pltpu.VMEM((_CH,), jnp.int32),
            pltpu.VMEM((_CH,), jnp.float32),
            pltpu.VMEM((_CH, _D), jnp.float32),
            pltpu.VMEM_SHARED((_NPAD, _D), jnp.float32),
            pltpu.VMEM_SHARED((_NPAD,), jnp.float32),
            pltpu.SemaphoreType.DMA,
            pltpu.SemaphoreType.DMA,
        ],
        compiler_params=_sc_compiler_params(),
    )
    def edge_kernel(src_r, dst_r, asrc_r, adst_r, m_r, h_r, zr_r, zd_r,
                    accp_r, denp_r,
                    asrc_t, adst_t, m_t, srcb, dstb, pb, rows_t,
                    acc_sp, den_sp, gsem, dsem):
        c = lax.axis_index("c")
        s = lax.axis_index("s")
        w = c * _NS + s

        pltpu.sync_copy(asrc_r, asrc_t)
        pltpu.sync_copy(adst_r, adst_t)
        pltpu.sync_copy(m_r, m_t)

        pltpu.sync_copy(zr_r, acc_sp.at[pl.ds(s * _ZR, _ZR)])
        pltpu.sync_copy(zd_r, den_sp.at[pl.ds(s * _ZR, _ZR)])
        plsc.subcore_barrier()

        mv = m_t[...]

        @pl.loop(w, _NB, step=_NW)
        def _chunk(blk):
            base = blk * _CH
            pltpu.sync_copy(src_r.at[pl.ds(base, _CH)], srcb)
            pltpu.sync_copy(dst_r.at[pl.ds(base, _CH)], dstb)

            gcp = pltpu.async_copy(h_r.at[srcb], rows_t, gsem)

            @pl.loop(0, _CH, step=16)
            def _group(j):
                si = srcb[pl.ds(j, 16)]
                di = dstb[pl.ds(j, 16)]
                a = plsc.load_gather(asrc_t, [si])
                b = plsc.load_gather(adst_t, [di])
                e = a + b
                e = jnp.where(e >= 0.0, e, e * 0.2)
                pb[pl.ds(j, 16)] = jnp.exp(e - mv)

            dcp = pltpu.async_copy(pb, den_sp.at[dstb], dsem, add=True)

            gcp.wait()

            @pl.loop(0, _CH, step=8)
            def _scale(r):
                for u in range(8):
                    pv = plsc.load_gather(pb, [jnp.broadcast_to(r + u, (16,))])
                    for k in range(0, _D, 16):
                        rows_t[r + u, pl.ds(k, 16)] = \
                            rows_t[r + u, pl.ds(k, 16)] * pv

            pltpu.sync_copy(rows_t, acc_sp.at[dstb], add=True)
            dcp.wait()

        plsc.subcore_barrier()

        @pl.when(s == 0)
        def _writeback():
            pltpu.sync_copy(acc_sp.at[pl.ds(0, _N)], accp_r.at[c])
            pltpu.sync_copy(den_sp, denp_r.at[c])

    return edge_kernel(src, dst, asrc, adst, mvec, h, zrows, zden)


def _attn_tail(h, a_s, a_d, as_o, ad_o, m_o):
    asr = jnp.sum(h * a_s[None, :], axis=1)
    adr = jnp.sum(h * a_d[None, :], axis=1)
    as_o[...] = asr
    ad_o[...] = adr
    m = jnp.max(asr) + jnp.max(adr)
    m = jnp.where(m >= 0.0, m, m * 0.2)
    m_o[...] = jnp.broadcast_to(m, (16,))


_TC_OUT_TYPES = [
    jax.ShapeDtypeStruct((_N, _D), jnp.float32),
    jax.ShapeDtypeStruct((_N,), jnp.float32),
    jax.ShapeDtypeStruct((_N,), jnp.float32),
    jax.ShapeDtypeStruct((16,), jnp.float32),
]


def _tc_in(x, W1, a_s, a_d):
    def body(x_r, w_r, as_r, ad_r, h_o, as_o, ad_o, m_o):
        h = lax.dot_general(x_r[...], w_r[...], (((1,), (1,)), ((), ())),
                            preferred_element_type=jnp.float32)
        h_o[...] = h
        _attn_tail(h, as_r[...], ad_r[...], as_o, ad_o, m_o)

    return pl.pallas_call(body, out_shape=_TC_OUT_TYPES)(x, W1, a_s, a_d)


def _norm_relu(acc_r, den_r, b):
    den = den_r[0, : _N] + den_r[1, : _N] + 1e-16
    hs = acc_r[0] + acc_r[1]
    h = hs / den[:, None] + b[None, :]
    return jnp.maximum(h, 0.0)


def _tc_mid(accp, denp, b, W, a_s, a_d):
    def body(acc_r, den_r, b_r, w_r, as_r, ad_r, h_o, as_o, ad_o, m_o):
        h1 = _norm_relu(acc_r, den_r, b_r[...])
        h = lax.dot_general(h1, w_r[...], (((1,), (1,)), ((), ())),
                            preferred_element_type=jnp.float32)
        h_o[...] = h
        _attn_tail(h, as_r[...], ad_r[...], as_o, ad_o, m_o)

    return pl.pallas_call(body, out_shape=_TC_OUT_TYPES)(
        accp, denp, b, W, a_s, a_d)


def _tc_out(accp, denp, b, fsW, fsb):
    def body(acc_r, den_r, b_r, w_r, wb_r, o_r):
        h = _norm_relu(acc_r, den_r, b_r[...])
        o_r[...] = lax.dot_general(h, w_r[...], (((1,), (1,)), ((), ())),
                                   preferred_element_type=jnp.float32) \
            + wb_r[...][None, :]

    return pl.pallas_call(
        body,
        out_shape=[jax.ShapeDtypeStruct((_N, 16), jnp.float32)],
    )(accp, denp, b, fsW, fsb)[0]


def kernel(x, edge_index, W1, att_src1, att_dst1, b1,
           W2, att_src2, att_dst2, b2, fW, fb, sW, sb):
    src = jnp.concatenate([edge_index[0].astype(jnp.int32),
                           jnp.zeros((_EP - _E,), jnp.int32)])
    dst = jnp.concatenate([edge_index[1].astype(jnp.int32),
                           jnp.full((_EP - _E,), _TRASH, jnp.int32)])
    zrows = jnp.zeros((_ZR, _D), jnp.float32)
    zden = jnp.zeros((_ZR,), jnp.float32)

    h1, as1, ad1, m1 = _tc_in(x, W1, att_src1, att_dst1)
    acc1, den1 = _sc_edge_pass(src, dst, as1, ad1, m1, h1, zrows, zden)
    h2, as2, ad2, m2 = _tc_mid(acc1, den1, b1, W2, att_src2, att_dst2)
    acc2, den2 = _sc_edge_pass(src, dst, as2, ad2, m2, h2, zrows, zden)

    fsW = jnp.concatenate([fW, sW, jnp.zeros((6, _D), jnp.float32)], axis=0)
    fsb = jnp.concatenate([fb, sb, jnp.zeros((6,), jnp.float32)], axis=0)
    out = _tc_out(acc2, den2, b2, fsW, fsb)
    return out[:, :3], out[:, 3:10]


# speedup vs baseline: 1.7414x; 1.0212x over previous
import dataclasses

import jax
import jax.numpy as jnp
from jax import lax
from jax.experimental import pallas as pl
from jax.experimental.pallas import tpu as pltpu
from jax.experimental.pallas import tpu_sc as plsc

_N = 10000
_E = 320000
_D = 128
_NC = 2
_NS = 16
_NW = _NC * _NS
_CH = 208
_EP = 320320
_NB = _EP // _CH
_NPAD = 10240
_ZR = _NPAD // _NS
_TRASH = _NPAD - 2


def _sc_compiler_params():
    cp = pltpu.CompilerParams()
    if "needs_layout_passes" in pltpu.CompilerParams.__dataclass_fields__:
        cp = dataclasses.replace(cp, needs_layout_passes=False)
    return cp


def _sc_edge_pass(eil, asrc, adst, mvec, h, zrows, zden):
    mesh = plsc.VectorSubcoreMesh(core_axis_name="c", subcore_axis_name="s")

    @pl.kernel(
        out_type=[
            jax.ShapeDtypeStruct((_NC, _N, _D), jnp.float32),
            jax.ShapeDtypeStruct((_NC, _NPAD), jnp.float32),
        ],
        mesh=mesh,
        scratch_types=[
            pltpu.VMEM((_N,), jnp.float32),
            pltpu.VMEM((_N,), jnp.float32),
            pltpu.VMEM((16,), jnp.float32),
            pltpu.VMEM((2 * _CH,), jnp.int32),
            pltpu.VMEM((_CH,), jnp.int32),
            pltpu.VMEM((_CH,), jnp.float32),
            pltpu.VMEM((_CH, _D), jnp.float32),
            pltpu.VMEM_SHARED((_NPAD, _D), jnp.float32),
            pltpu.VMEM_SHARED((_NPAD,), jnp.float32),
            pltpu.SemaphoreType.DMA,
            pltpu.SemaphoreType.DMA,
        ],
        compiler_params=_sc_compiler_params(),
    )
    def edge_kernel(src_r, asrc_r, adst_r, m_r, h_r, zr_r, zd_r,
                    accp_r, denp_r,
                    asrc_t, adst_t, m_t, srcb, dstb, pb, rows_t,
                    acc_sp, den_sp, gsem, dsem):
        c = lax.axis_index("c")
        s = lax.axis_index("s")
        w = c * _NS + s

        pltpu.sync_copy(asrc_r, asrc_t)
        pltpu.sync_copy(adst_r, adst_t)
        pltpu.sync_copy(m_r, m_t)

        pltpu.sync_copy(zr_r, acc_sp.at[pl.ds(s * _ZR, _ZR)])
        pltpu.sync_copy(zd_r, den_sp.at[pl.ds(s * _ZR, _ZR)])
        plsc.subcore_barrier()

        mv = m_t[...]

        @pl.loop(w, _NB, step=_NW)
        def _chunk(blk):
            pltpu.sync_copy(src_r.at[pl.ds(blk * 2 * _CH, 2 * _CH)], srcb)

            @pl.loop(0, _CH, step=16)
            def _dmove(j):
                dstb[pl.ds(j, 16)] = srcb[pl.ds(_CH + j, 16)]

            gcp = pltpu.async_copy(h_r.at[srcb.at[pl.ds(0, _CH)]], rows_t,
                                   gsem)

            @pl.loop(0, _CH, step=16)
            def _group(j):
                si = srcb[pl.ds(j, 16)]
                di = dstb[pl.ds(j, 16)]
                a = plsc.load_gather(asrc_t, [si])
                b = plsc.load_gather(adst_t, [di])
                e = a + b
                e = jnp.where(e >= 0.0, e, e * 0.2)
                pb[pl.ds(j, 16)] = jnp.exp(e - mv)

            dcp = pltpu.async_copy(pb, den_sp.at[dstb], dsem, add=True)

            gcp.wait()

            @pl.loop(0, _CH, step=8)
            def _scale(r):
                for u in range(8):
                    pv = plsc.load_gather(pb, [jnp.broadcast_to(r + u, (16,))])
                    for k in range(0, _D, 16):
                        rows_t[r + u, pl.ds(k, 16)] = \
                            rows_t[r + u, pl.ds(k, 16)] * pv

            pltpu.sync_copy(rows_t, acc_sp.at[dstb], add=True)
            dcp.wait()

        plsc.subcore_barrier()

        @pl.when(s == 0)
        def _writeback():
            pltpu.sync_copy(acc_sp.at[pl.ds(0, _N)], accp_r.at[c])
            pltpu.sync_copy(den_sp, denp_r.at[c])

    return edge_kernel(eil, asrc, adst, mvec, h, zrows, zden)


def _attn_tail(h, a_s, a_d, as_o, ad_o, m_o):
    asr = jnp.sum(h * a_s[None, :], axis=1)
    adr = jnp.sum(h * a_d[None, :], axis=1)
    as_o[...] = asr
    ad_o[...] = adr
    m = jnp.max(asr) + jnp.max(adr)
    m = jnp.where(m >= 0.0, m, m * 0.2)
    m_o[...] = jnp.broadcast_to(m, (16,))


_TC_OUT_TYPES = [
    jax.ShapeDtypeStruct((_N, _D), jnp.float32),
    jax.ShapeDtypeStruct((_N,), jnp.float32),
    jax.ShapeDtypeStruct((_N,), jnp.float32),
    jax.ShapeDtypeStruct((16,), jnp.float32),
]


def _tc_in(x, W1, a_s, a_d):
    def body(x_r, w_r, as_r, ad_r, h_o, as_o, ad_o, m_o):
        h = lax.dot_general(x_r[...], w_r[...], (((1,), (1,)), ((), ())),
                            preferred_element_type=jnp.float32)
        h_o[...] = h
        _attn_tail(h, as_r[...], ad_r[...], as_o, ad_o, m_o)

    return pl.pallas_call(body, out_shape=_TC_OUT_TYPES)(x, W1, a_s, a_d)


def _norm_relu(acc_r, den_r, b):
    den = den_r[0, : _N] + den_r[1, : _N] + 1e-16
    hs = acc_r[0] + acc_r[1]
    h = hs / den[:, None] + b[None, :]
    return jnp.maximum(h, 0.0)


def _tc_mid(accp, denp, b, W, a_s, a_d):
    def body(acc_r, den_r, b_r, w_r, as_r, ad_r, h_o, as_o, ad_o, m_o):
        h1 = _norm_relu(acc_r, den_r, b_r[...])
        h = lax.dot_general(h1, w_r[...], (((1,), (1,)), ((), ())),
                            preferred_element_type=jnp.float32)
        h_o[...] = h
        _attn_tail(h, as_r[...], ad_r[...], as_o, ad_o, m_o)

    return pl.pallas_call(body, out_shape=_TC_OUT_TYPES)(
        accp, denp, b, W, a_s, a_d)


def _tc_out(accp, denp, b, fsW, fsb):
    def body(acc_r, den_r, b_r, w_r, wb_r, o_r):
        h = _norm_relu(acc_r, den_r, b_r[...])
        o_r[...] = lax.dot_general(h, w_r[...], (((1,), (1,)), ((), ())),
                                   preferred_element_type=jnp.float32) \
            + wb_r[...][None, :]

    return pl.pallas_call(
        body,
        out_shape=[jax.ShapeDtypeStruct((_N, 16), jnp.float32)],
    )(accp, denp, b, fsW, fsb)[0]


def kernel(x, edge_index, W1, att_src1, att_dst1, b1,
           W2, att_src2, att_dst2, b2, fW, fb, sW, sb):
    src = jnp.concatenate([edge_index[0].astype(jnp.int32),
                           jnp.zeros((_EP - _E,), jnp.int32)])
    dst = jnp.concatenate([edge_index[1].astype(jnp.int32),
                           jnp.full((_EP - _E,), _TRASH, jnp.int32)])
    eil = jnp.stack([src.reshape(_NB, _CH), dst.reshape(_NB, _CH)],
                    axis=1).reshape(-1)
    zrows = jnp.zeros((_ZR, _D), jnp.float32)
    zden = jnp.zeros((_ZR,), jnp.float32)

    h1, as1, ad1, m1 = _tc_in(x, W1, att_src1, att_dst1)
    acc1, den1 = _sc_edge_pass(eil, as1, ad1, m1, h1, zrows, zden)
    h2, as2, ad2, m2 = _tc_mid(acc1, den1, b1, W2, att_src2, att_dst2)
    acc2, den2 = _sc_edge_pass(eil, as2, ad2, m2, h2, zrows, zden)

    fsW = jnp.concatenate([fW, sW, jnp.zeros((6, _D), jnp.float32)], axis=0)
    fsb = jnp.concatenate([fb, sb, jnp.zeros((6,), jnp.float32)], axis=0)
    out = _tc_out(acc2, den2, b2, fsW, fsb)
    return out[:, :3], out[:, 3:10]
